# baseline (speedup 1.0000x reference)
import functools

import jax
import jax.numpy as jnp
from jax import lax
from jax.experimental import pallas as pl
from jax.experimental.pallas import tpu as pltpu
from jax.experimental.pallas import tpu_sc as plsc

_NBUF = 2


def _gather_kernel(n_batch, seq, hidden, num_cores, num_subcores):
    num_workers = num_cores * num_subcores
    rows_per_w = n_batch // num_workers
    chunk = seq
    per_w = rows_per_w * chunk
    n_iters = rows_per_w // _NBUF

    mesh = plsc.VectorSubcoreMesh(core_axis_name="c", subcore_axis_name="s")

    @functools.partial(
        pl.kernel,
        mesh=mesh,
        compiler_params=pltpu.CompilerParams(
            use_tc_tiling_on_sc=False, needs_layout_passes=False),
        out_type=jax.ShapeDtypeStruct((n_batch, seq, 2 * hidden),
                                      jnp.bfloat16),
        scratch_types=[
            pltpu.VMEM((per_w,), jnp.int32),
            pltpu.VMEM((_NBUF, chunk, hidden), jnp.int32),
            pltpu.VMEM((_NBUF, chunk, 2 * hidden), jnp.bfloat16),
            [pltpu.SemaphoreType.DMA] * _NBUF,
            [pltpu.SemaphoreType.DMA] * _NBUF,
        ],
    )
    def body(idx_hbm, table_hbm, out_bf16, idx_v, bufi, bufo, gsems, ssems):
        wid = lax.axis_index("s") * num_cores + lax.axis_index("c")
        base = wid * per_w
        row0 = wid * rows_per_w
        pltpu.sync_copy(idx_hbm.at[pl.ds(base, per_w)], idx_v)

        def start_gather(j, b):
            pltpu.async_copy(
                table_hbm.at[idx_v.at[pl.ds(j * chunk, chunk)]],
                bufi.at[b], gsems[b])

        def wait_gather(b):
            pltpu.make_async_copy(
                table_hbm.at[idx_v.at[pl.ds(0, chunk)]],
                bufi.at[b], gsems[b]).wait()

        def wait_store(b):
            pltpu.make_async_copy(
                bufo.at[b], out_bf16.at[row0], ssems[b]).wait()

        def convert(b):
            src = bufi.at[b]
            dst = bufo.at[b]

            def rr(q, c):
                for k in range(hidden // 16):
                    v = src[q, pl.ds(16 * k, 16)]
                    dst[q, pl.ds(32 * k, 32)] = plsc.bitcast(
                        v, jnp.bfloat16)
                return c

            lax.fori_loop(0, chunk, rr, 0)

        for b in range(_NBUF):
            start_gather(b, b)

        def outer(i, carry):
            for b in range(_NBUF):
                j = i * _NBUF + b
                wait_gather(b)

                @pl.when(i > 0)
                def _():
                    wait_store(b)

                convert(b)

                @pl.when(i < n_iters - 1)
                def _():
                    start_gather((i + 1) * _NBUF + b, b)

                pltpu.make_async_copy(
                    bufo.at[b], out_bf16.at[row0 + j], ssems[b]).start()

            return carry

        lax.fori_loop(0, n_iters, outer, 0)
        for b in range(_NBUF):
            wait_store(b)

    return body


def kernel(x, table):
    b, l = x.shape
    vocab, hidden = table.shape
    info = plsc.get_sparse_core_info()
    idx = x.reshape(b * l).astype(jnp.int32)
    table_i32 = jax.lax.bitcast_convert_type(
        table.reshape(vocab, hidden // 2, 2), jnp.int32)
    fn = _gather_kernel(b, l, hidden // 2,
                        info.num_cores, info.num_subcores)
    return fn(idx, table_i32)


# speedup vs baseline: 1.2955x; 1.2955x over previous
import functools

import jax
import jax.numpy as jnp
from jax import lax
from jax.experimental import pallas as pl
from jax.experimental.pallas import tpu as pltpu
from jax.experimental.pallas import tpu_sc as plsc

_NBUF = 2
_TCHUNK = 125


def _gather_kernel(n_batch, seq, hidden2, vocab, num_cores, num_subcores):
    hidden = hidden2 // 2
    num_workers = num_cores * num_subcores
    rows_per_w = n_batch // num_workers
    chunk = seq
    per_w = rows_per_w * chunk
    n_iters = rows_per_w // _NBUF
    trows_per_w = vocab // num_subcores
    t_iters = trows_per_w // _TCHUNK // _NBUF

    mesh = plsc.VectorSubcoreMesh(core_axis_name="c", subcore_axis_name="s")

    @functools.partial(
        pl.kernel,
        mesh=mesh,
        compiler_params=pltpu.CompilerParams(
            use_tc_tiling_on_sc=False, needs_layout_passes=False),
        out_type=jax.ShapeDtypeStruct((n_batch, seq, hidden2), jnp.bfloat16),
        scratch_types=[
            pltpu.HBM((num_cores, vocab, hidden), jnp.int32),
            pltpu.VMEM((rows_per_w, chunk), jnp.int32),
            pltpu.VMEM((_NBUF, chunk, hidden), jnp.int32),
            pltpu.VMEM((_NBUF, chunk, hidden2), jnp.bfloat16),
            [pltpu.SemaphoreType.DMA] * _NBUF,
            [pltpu.SemaphoreType.DMA] * _NBUF,
            [pltpu.SemaphoreType.DMA] * _NBUF,
        ],
    )
    def body(idx_hbm, table_hbm, out_bf16, tab32, idx_v, bufi, bufo,
             gsems, ssems, isems):
        ci = lax.axis_index("c")
        sid = lax.axis_index("s")
        wid = sid * num_cores + ci
        row0 = wid * rows_per_w

        tbase = sid * trows_per_w
        my_tab = tab32.at[ci]

        def t_load(step, b):
            pltpu.async_copy(
                table_hbm.at[pl.ds(tbase + step * _TCHUNK, _TCHUNK)],
                bufo.at[b, pl.ds(0, _TCHUNK)], isems[b])

        def t_wait_load(b):
            pltpu.make_async_copy(
                table_hbm.at[pl.ds(0, _TCHUNK)],
                bufo.at[b, pl.ds(0, _TCHUNK)], isems[b]).wait()

        def t_wait_store(b):
            pltpu.make_async_copy(
                bufi.at[b, pl.ds(0, _TCHUNK)],
                my_tab.at[pl.ds(0, _TCHUNK)], ssems[b]).wait()

        def t_convert(b):
            src = bufo.at[b]
            dst = bufi.at[b]

            def rr(q, c):
                for k in range(hidden2 // 32):
                    v = src[q, pl.ds(32 * k, 32)]
                    dst[q, pl.ds(16 * k, 16)] = plsc.bitcast(v, jnp.int32)
                return c

            lax.fori_loop(0, _TCHUNK, rr, 0)

        for b in range(_NBUF):
            t_load(b, b)

        def t_outer(i, carry):
            for b in range(_NBUF):
                step = i * _NBUF + b
                t_wait_load(b)

                @pl.when(i > 0)
                def _():
                    t_wait_store(b)

                t_convert(b)

                @pl.when(i < t_iters - 1)
                def _():
                    t_load((i + 1) * _NBUF + b, b)

                pltpu.make_async_copy(
                    bufi.at[b, pl.ds(0, _TCHUNK)],
                    my_tab.at[pl.ds(tbase + step * _TCHUNK, _TCHUNK)],
                    ssems[b]).start()

            return carry

        lax.fori_loop(0, t_iters, t_outer, 0)
        for b in range(_NBUF):
            t_wait_store(b)
        pltpu.sync_copy(idx_hbm.at[pl.ds(row0, rows_per_w)], idx_v)
        plsc.subcore_barrier()

        def start_gather(j, b):
            pltpu.async_copy(
                my_tab.at[idx_v.at[j]], bufi.at[b, pl.ds(0, chunk)],
                gsems[b])

        def wait_gather(b):
            pltpu.make_async_copy(
                my_tab.at[idx_v.at[0]], bufi.at[b, pl.ds(0, chunk)],
                gsems[b]).wait()

        def wait_store(b):
            pltpu.make_async_copy(
                bufo.at[b, pl.ds(0, chunk)], out_bf16.at[row0],
                ssems[b]).wait()

        def convert(b):
            src = bufi.at[b]
            dst = bufo.at[b]

            def rr(q, c):
                for k in range(hidden // 16):
                    v = src[q, pl.ds(16 * k, 16)]
                    dst[q, pl.ds(32 * k, 32)] = plsc.bitcast(
                        v, jnp.bfloat16)
                return c

            lax.fori_loop(0, chunk, rr, 0)

        for b in range(_NBUF):
            start_gather(b, b)

        def outer(i, carry):
            for b in range(_NBUF):
                j = i * _NBUF + b
                wait_gather(b)

                @pl.when(i > 0)
                def _():
                    wait_store(b)

                convert(b)

                @pl.when(i < n_iters - 1)
                def _():
                    start_gather((i + 1) * _NBUF + b, b)

                pltpu.make_async_copy(
                    bufo.at[b, pl.ds(0, chunk)], out_bf16.at[row0 + j],
                    ssems[b]).start()

            return carry

        lax.fori_loop(0, n_iters, outer, 0)
        for b in range(_NBUF):
            wait_store(b)

    return body


def kernel(x, table):
    b, l = x.shape
    vocab, hidden = table.shape
    info = plsc.get_sparse_core_info()
    fn = _gather_kernel(b, l, hidden, vocab,
                        info.num_cores, info.num_subcores)
    return fn(x.astype(jnp.int32), table)


# speedup vs baseline: 1.3047x; 1.0071x over previous
import functools

import jax
import jax.numpy as jnp
from jax import lax
from jax.experimental import pallas as pl
from jax.experimental.pallas import tpu as pltpu
from jax.experimental.pallas import tpu_sc as plsc

_NBUF = 4
_TNBUF = 2
_TCHUNK = 125


def _gather_kernel(n_batch, seq, hidden2, vocab, num_cores, num_subcores):
    hidden = hidden2 // 2
    num_workers = num_cores * num_subcores
    rows_per_w = n_batch // num_workers
    chunk = seq
    per_w = rows_per_w * chunk
    n_iters = rows_per_w // _NBUF
    trows_per_w = vocab // num_subcores
    t_iters = trows_per_w // _TCHUNK // _TNBUF

    mesh = plsc.VectorSubcoreMesh(core_axis_name="c", subcore_axis_name="s")

    @functools.partial(
        pl.kernel,
        mesh=mesh,
        compiler_params=pltpu.CompilerParams(
            use_tc_tiling_on_sc=False, needs_layout_passes=False),
        out_type=jax.ShapeDtypeStruct((n_batch, seq, hidden2), jnp.bfloat16),
        scratch_types=[
            pltpu.HBM((num_cores, vocab, hidden), jnp.int32),
            pltpu.VMEM((rows_per_w, chunk), jnp.int32),
            pltpu.VMEM((_NBUF, chunk, hidden), jnp.int32),
            pltpu.VMEM((_NBUF, chunk, hidden2), jnp.bfloat16),
            [pltpu.SemaphoreType.DMA] * _NBUF,
            [pltpu.SemaphoreType.DMA] * _NBUF,
            [pltpu.SemaphoreType.DMA] * _NBUF,
        ],
    )
    def body(idx_hbm, table_hbm, out_bf16, tab32, idx_v, bufi, bufo,
             gsems, ssems, isems):
        ci = lax.axis_index("c")
        sid = lax.axis_index("s")
        wid = sid * num_cores + ci
        row0 = wid * rows_per_w

        tbase = sid * trows_per_w
        my_tab = tab32.at[ci]

        def t_load(step, b):
            pltpu.async_copy(
                table_hbm.at[pl.ds(tbase + step * _TCHUNK, _TCHUNK)],
                bufo.at[b, pl.ds(0, _TCHUNK)], isems[b])

        def t_wait_load(b):
            pltpu.make_async_copy(
                table_hbm.at[pl.ds(0, _TCHUNK)],
                bufo.at[b, pl.ds(0, _TCHUNK)], isems[b]).wait()

        def t_wait_store(b):
            pltpu.make_async_copy(
                bufi.at[b, pl.ds(0, _TCHUNK)],
                my_tab.at[pl.ds(0, _TCHUNK)], ssems[b]).wait()

        def t_convert(b):
            src = bufo.at[b]
            dst = bufi.at[b]

            def rr(q, c):
                for k in range(hidden2 // 32):
                    v = src[q, pl.ds(32 * k, 32)]
                    dst[q, pl.ds(16 * k, 16)] = plsc.bitcast(v, jnp.int32)
                return c

            lax.fori_loop(0, _TCHUNK, rr, 0)

        for b in range(_TNBUF):
            t_load(b, b)

        def t_outer(i, carry):
            for b in range(_TNBUF):
                step = i * _TNBUF + b
                t_wait_load(b)

                @pl.when(i > 0)
                def _():
                    t_wait_store(b)

                t_convert(b)

                @pl.when(i < t_iters - 1)
                def _():
                    t_load((i + 1) * _TNBUF + b, b)

                pltpu.make_async_copy(
                    bufi.at[b, pl.ds(0, _TCHUNK)],
                    my_tab.at[pl.ds(tbase + step * _TCHUNK, _TCHUNK)],
                    ssems[b]).start()

            return carry

        lax.fori_loop(0, t_iters, t_outer, 0)
        for b in range(_TNBUF):
            t_wait_store(b)
        pltpu.sync_copy(idx_hbm.at[pl.ds(row0, rows_per_w)], idx_v)
        plsc.subcore_barrier()

        def start_gather(j, b):
            pltpu.async_copy(
                my_tab.at[idx_v.at[j]], bufi.at[b, pl.ds(0, chunk)],
                gsems[b])

        def wait_gather(b):
            pltpu.make_async_copy(
                my_tab.at[idx_v.at[0]], bufi.at[b, pl.ds(0, chunk)],
                gsems[b]).wait()

        def wait_store(b):
            pltpu.make_async_copy(
                bufo.at[b, pl.ds(0, chunk)], out_bf16.at[row0],
                ssems[b]).wait()

        def convert(b):
            src = bufi.at[b]
            dst = bufo.at[b]

            def rr(q, c):
                for k in range(hidden // 16):
                    v = src[q, pl.ds(16 * k, 16)]
                    dst[q, pl.ds(32 * k, 32)] = plsc.bitcast(
                        v, jnp.bfloat16)
                return c

            lax.fori_loop(0, chunk, rr, 0)

        for b in range(_NBUF):
            start_gather(b, b)

        def outer(i, carry):
            for b in range(_NBUF):
                j = i * _NBUF + b
                wait_gather(b)

                @pl.when(i > 0)
                def _():
                    wait_store(b)

                convert(b)

                @pl.when(i < n_iters - 1)
                def _():
                    start_gather((i + 1) * _NBUF + b, b)

                pltpu.make_async_copy(
                    bufo.at[b, pl.ds(0, chunk)], out_bf16.at[row0 + j],
                    ssems[b]).start()

            return carry

        lax.fori_loop(0, n_iters, outer, 0)
        for b in range(_NBUF):
            wait_store(b)

    return body


def kernel(x, table):
    b, l = x.shape
    vocab, hidden = table.shape
    info = plsc.get_sparse_core_info()
    fn = _gather_kernel(b, l, hidden, vocab,
                        info.num_cores, info.num_subcores)
    return fn(x.astype(jnp.int32), table)
